# baseline (speedup 1.0000x reference)
import functools

import jax
import jax.numpy as jnp
import numpy as np
from jax.experimental import pallas as pl
from jax.experimental.pallas import tpu as pltpu
from jax.experimental.pallas import tpu_sc as plsc

VOCAB = 1000
PV = 1024
NTOK = 65536
TB = 512

_TINY = np.float32(np.finfo(np.float32).tiny)
_K0 = 0
_K1 = 42
_KS2 = np.uint32(_K0 ^ _K1 ^ 0x1BD11BDA)
_ROT = ((13, 15, 26, 6), (17, 29, 16, 24))

_NC, _NS, _NL = 2, 16, 16
_NW = _NC * _NS

SC_TOK = 24576
TOK0 = NTOK - SC_TOK
NSTEPS_A = TOK0 // TB
NSTEPS_B = SC_TOK // TB

_TPW = SC_TOK // _NW
_RG = 8
_NG = _TPW // _RG
_GELEM = _RG * PV
_GCH = _GELEM // _NL
_UNROLL = 4


def _rotl(x, r):
    return (x << jnp.uint32(r)) | (x >> jnp.uint32(32 - r))


def _threefry_bits(x1):
    ks = (np.uint32(_K0), np.uint32(_K1), _KS2)
    x0 = x1
    x1 = _rotl(x1, _ROT[0][0]) ^ x0
    for r in _ROT[0][1:]:
        x0 = x0 + x1
        x1 = _rotl(x1, r)
        x1 = x1 ^ x0
    x0 = x0 + ks[1]
    x1 = x1 + np.uint32(ks[2] + np.uint32(1))
    for i in range(1, 5):
        for r in _ROT[i % 2]:
            x0 = x0 + x1
            x1 = _rotl(x1, r)
            x1 = x1 ^ x0
        x0 = x0 + ks[(i + 1) % 3]
        x1 = x1 + np.uint32(ks[(i + 2) % 3] + np.uint32(i + 1))
    return x0 ^ x1


def _make_body(from_sc_bits):
    def _body(src_ref, mb_ref, tabhi_ref, tablo_ref, yp_ref, loss_ref,
              acc_ref):
        step = pl.program_id(0)
        col = jax.lax.broadcasted_iota(jnp.int32, (TB, PV), 1)

        @pl.when(step == 0)
        def _init():
            acc_ref[0, 0] = 0.0

        mb = mb_ref[0]

        onehot = (col == mb).astype(jnp.bfloat16)
        logits = (jnp.dot(onehot, tabhi_ref[...],
                          preferred_element_type=jnp.float32)
                  + jnp.dot(onehot, tablo_ref[...],
                            preferred_element_type=jnp.float32))

        if from_sc_bits:
            bits = src_ref[0]
        else:
            bits = _threefry_bits(
                src_ref[...] + (step * (TB * VOCAB)).astype(jnp.uint32))

        fb = (bits >> jnp.uint32(9)) | jnp.uint32(0x3F800000)
        f = jax.lax.bitcast_convert_type(fb, jnp.float32) - jnp.float32(1.0)
        u = jnp.maximum(_TINY, f)
        g = -jnp.log(-jnp.log(u))

        z = logits + g
        zmax = jnp.max(z, axis=1, keepdims=True)
        win = jnp.min(jnp.where(z == zmax, col, PV), axis=1, keepdims=True)
        yp_ref[0] = win

        lmax = jnp.max(logits, axis=1, keepdims=True)
        s = jnp.sum(jnp.exp(logits - lmax), axis=1, keepdims=True)
        lse_tok = lmax + jnp.log(s)
        acc_ref[0, 0] += jnp.sum(lse_tok)
        loss_ref[...] = jnp.full((1, 1), acc_ref[0, 0], jnp.float32)

    return _body


def _tc_sample(src, mb, tab_hi, tab_lo, nsteps, from_sc_bits):
    src_spec = (pl.BlockSpec((1, TB, PV), lambda i: (i, 0, 0))
                if from_sc_bits else
                pl.BlockSpec((TB, PV), lambda i: (0, 0)))
    return pl.pallas_call(
        _make_body(from_sc_bits),
        grid=(nsteps,),
        in_specs=[
            src_spec,
            pl.BlockSpec((1, TB, 1), lambda i: (i, 0, 0)),
            pl.BlockSpec((PV, PV), lambda i: (0, 0)),
            pl.BlockSpec((PV, PV), lambda i: (0, 0)),
        ],
        out_specs=[
            pl.BlockSpec((1, TB, 1), lambda i: (i, 0, 0)),
            pl.BlockSpec((1, 1), lambda i: (0, 0)),
        ],
        out_shape=[
            jax.ShapeDtypeStruct((nsteps, TB, 1), jnp.int32),
            jax.ShapeDtypeStruct((1, 1), jnp.float32),
        ],
        scratch_shapes=[
            pltpu.SMEM((1, 1), jnp.float32),
        ],
    )(src, mb, tab_hi, tab_lo)


def _bits_body(out_hbm, buf_v):
    wid = jax.lax.axis_index("s") * _NC + jax.lax.axis_index("c")
    tok_base = TOK0 + wid * _TPW
    lane = jax.lax.iota(jnp.uint32, _NL)

    def _group(o, carry):
        def _chunk(j, c2):
            for k in range(_UNROLL):
                ci = j * _UNROLL + k
                t_rel = ci >> 6
                c0 = (ci & 63) << 4
                cnt0 = ((tok_base + o * _RG + t_rel) * VOCAB + c0
                        + _K1)
                x1 = lane + cnt0.astype(jnp.uint32)
                buf_v[pl.ds(ci * _NL, _NL)] = _threefry_bits(x1)
            return c2

        jax.lax.fori_loop(0, _GCH // _UNROLL, _chunk, 0)
        pltpu.sync_copy(
            buf_v,
            out_hbm.at[pl.ds((wid * _TPW + o * _RG) * PV, _GELEM)])
        return carry

    jax.lax.fori_loop(0, _NG, _group, 0)


_sc_bits = functools.partial(
    pl.kernel,
    _bits_body,
    out_type=jax.ShapeDtypeStruct((SC_TOK * PV,), jnp.uint32),
    mesh=plsc.VectorSubcoreMesh(core_axis_name="c", subcore_axis_name="s",
                                num_cores=_NC, num_subcores=_NS),
    scratch_types=[
        pltpu.VMEM((_GELEM,), jnp.uint32),
    ],
)()


_BPW = NTOK // _NW
_CH = _BPW // _NL


def _tgt_body(mb_hbm, y_hbm, tabf_hbm, out_hbm, mb_v, y_v, idx_v, val_v,
              acc_v, sem):
    wid = jax.lax.axis_index("s") * _NC + jax.lax.axis_index("c")
    base = wid * _BPW
    pltpu.sync_copy(mb_hbm.at[pl.ds(base, _BPW)], mb_v)
    pltpu.sync_copy(y_hbm.at[pl.ds(base, _BPW)], y_v)

    def _mk_idx(i, carry):
        sl = pl.ds(i * _NL, _NL)
        idx_v[sl] = mb_v[sl] * VOCAB + y_v[sl]
        return carry

    jax.lax.fori_loop(0, _CH, _mk_idx, 0)
    pltpu.async_copy(tabf_hbm.at[idx_v], val_v, sem).wait()

    def _acc(i, acc):
        return acc + val_v[pl.ds(i * _NL, _NL)]

    acc_v[...] = jax.lax.fori_loop(0, _CH, _acc,
                                   jnp.zeros((_NL,), jnp.float32))
    pltpu.sync_copy(acc_v, out_hbm.at[wid])


_tgt_partials = functools.partial(
    pl.kernel,
    _tgt_body,
    out_type=jax.ShapeDtypeStruct((_NW, _NL), jnp.float32),
    mesh=plsc.VectorSubcoreMesh(core_axis_name="c", subcore_axis_name="s",
                                num_cores=_NC, num_subcores=_NS),
    scratch_types=[
        pltpu.VMEM((_BPW,), jnp.int32),
        pltpu.VMEM((_BPW,), jnp.int32),
        pltpu.VMEM((_BPW,), jnp.int32),
        pltpu.VMEM((_BPW,), jnp.float32),
        pltpu.VMEM((_NL,), jnp.float32),
        pltpu.SemaphoreType.DMA,
    ],
)()


@jax.jit
def kernel(mini_batch, y, embed_weight):
    mbf = mini_batch.reshape(NTOK)
    mb_a = mbf[:TOK0].reshape(NSTEPS_A, TB, 1)
    mb_b = mbf[TOK0:].reshape(NSTEPS_B, TB, 1)
    tab = jnp.pad(embed_weight, ((0, PV - VOCAB), (0, 0)))
    tab = jnp.pad(tab, ((0, 0), (0, PV - VOCAB)),
                  constant_values=-1e30)
    tab_hi = tab.astype(jnp.bfloat16)
    tab_lo = (tab - tab_hi.astype(jnp.float32)).astype(jnp.bfloat16)

    cnt_inv = (jnp.arange(TB, dtype=jnp.uint32)[:, None] * jnp.uint32(VOCAB)
               + jnp.arange(PV, dtype=jnp.uint32)[None, :]
               + jnp.uint32(_K1))

    bits_sc = _sc_bits()
    tgt_part = _tgt_partials(mbf, y.reshape(NTOK),
                             embed_weight.reshape(VOCAB * VOCAB))

    yp_a, lse_a = _tc_sample(cnt_inv, mb_a, tab_hi, tab_lo, NSTEPS_A,
                             from_sc_bits=False)
    yp_b, lse_b = _tc_sample(bits_sc.reshape(NSTEPS_B, TB, PV), mb_b,
                             tab_hi, tab_lo, NSTEPS_B, from_sc_bits=True)

    loss = ((lse_a[0, 0] + lse_b[0, 0]) - jnp.sum(tgt_part)) \
        * jnp.float32(1.0 / NTOK)
    yp = jnp.concatenate([yp_a.reshape(TOK0), yp_b.reshape(SC_TOK)])
    return yp, loss


# speedup vs baseline: 1.0501x; 1.0501x over previous
import functools

import jax
import jax.numpy as jnp
import numpy as np
from jax.experimental import pallas as pl
from jax.experimental.pallas import tpu as pltpu
from jax.experimental.pallas import tpu_sc as plsc

VOCAB = 1000
PV = 1024
NTOK = 65536
TB = 512

_TINY = np.float32(np.finfo(np.float32).tiny)
_K0 = 0
_K1 = 42
_KS2 = np.uint32(_K0 ^ _K1 ^ 0x1BD11BDA)
_ROT = ((13, 15, 26, 6), (17, 29, 16, 24))

_NC, _NS, _NL = 2, 16, 16
_NW = _NC * _NS

SC_TOK = 22528
TOK0 = NTOK - SC_TOK
NSTEPS_A = TOK0 // TB
NSTEPS_B = SC_TOK // TB

_TPW = SC_TOK // _NW
_RG = 8
_NG = _TPW // _RG
_GELEM = _RG * PV
_GCH = _GELEM // _NL
_UNROLL = 4


def _rotl(x, r):
    return (x << jnp.uint32(r)) | (x >> jnp.uint32(32 - r))


def _threefry_bits(x1):
    ks = (np.uint32(_K0), np.uint32(_K1), _KS2)
    x0 = x1
    x1 = _rotl(x1, _ROT[0][0]) ^ x0
    for r in _ROT[0][1:]:
        x0 = x0 + x1
        x1 = _rotl(x1, r)
        x1 = x1 ^ x0
    x0 = x0 + ks[1]
    x1 = x1 + np.uint32(ks[2] + np.uint32(1))
    for i in range(1, 5):
        for r in _ROT[i % 2]:
            x0 = x0 + x1
            x1 = _rotl(x1, r)
            x1 = x1 ^ x0
        x0 = x0 + ks[(i + 1) % 3]
        x1 = x1 + np.uint32(ks[(i + 2) % 3] + np.uint32(i + 1))
    return x0 ^ x1


def _make_body(from_sc_bits):
    def _body(src_ref, mb_ref, tabhi_ref, tablo_ref, yp_ref, loss_ref,
              acc_ref):
        step = pl.program_id(0)
        col = jax.lax.broadcasted_iota(jnp.int32, (TB, PV), 1)

        @pl.when(step == 0)
        def _init():
            acc_ref[0, 0] = 0.0

        mb = mb_ref[0]

        onehot = (col == mb).astype(jnp.bfloat16)
        logits = (jnp.dot(onehot, tabhi_ref[...],
                          preferred_element_type=jnp.float32)
                  + jnp.dot(onehot, tablo_ref[...],
                            preferred_element_type=jnp.float32))

        if from_sc_bits:
            bits = src_ref[0]
        else:
            bits = _threefry_bits(
                src_ref[...] + (step * (TB * VOCAB)).astype(jnp.uint32))

        fb = (bits >> jnp.uint32(9)) | jnp.uint32(0x3F800000)
        f = jax.lax.bitcast_convert_type(fb, jnp.float32) - jnp.float32(1.0)
        u = jnp.maximum(_TINY, f)
        g = -jnp.log(-jnp.log(u))

        z = logits + g
        zmax = jnp.max(z, axis=1, keepdims=True)
        win = jnp.min(jnp.where(z == zmax, col, PV), axis=1, keepdims=True)
        yp_ref[0] = win

        lmax = jnp.max(logits, axis=1, keepdims=True)
        s = jnp.sum(jnp.exp(logits - lmax), axis=1, keepdims=True)
        lse_tok = lmax + jnp.log(s)
        acc_ref[0, 0] += jnp.sum(lse_tok)
        loss_ref[...] = jnp.full((1, 1), acc_ref[0, 0], jnp.float32)

    return _body


def _tc_sample(src, mb, tab_hi, tab_lo, nsteps, from_sc_bits):
    src_spec = (pl.BlockSpec((1, TB, PV), lambda i: (i, 0, 0))
                if from_sc_bits else
                pl.BlockSpec((TB, PV), lambda i: (0, 0)))
    return pl.pallas_call(
        _make_body(from_sc_bits),
        grid=(nsteps,),
        in_specs=[
            src_spec,
            pl.BlockSpec((1, TB, 1), lambda i: (i, 0, 0)),
            pl.BlockSpec((PV, PV), lambda i: (0, 0)),
            pl.BlockSpec((PV, PV), lambda i: (0, 0)),
        ],
        out_specs=[
            pl.BlockSpec((1, TB, 1), lambda i: (i, 0, 0)),
            pl.BlockSpec((1, 1), lambda i: (0, 0)),
        ],
        out_shape=[
            jax.ShapeDtypeStruct((nsteps, TB, 1), jnp.int32),
            jax.ShapeDtypeStruct((1, 1), jnp.float32),
        ],
        scratch_shapes=[
            pltpu.SMEM((1, 1), jnp.float32),
        ],
    )(src, mb, tab_hi, tab_lo)


def _bits_body(out_hbm, buf_v):
    wid = jax.lax.axis_index("s") * _NC + jax.lax.axis_index("c")
    tok_base = TOK0 + wid * _TPW
    lane = jax.lax.iota(jnp.uint32, _NL)

    def _group(o, carry):
        def _chunk(j, c2):
            for k in range(_UNROLL):
                ci = j * _UNROLL + k
                t_rel = ci >> 6
                c0 = (ci & 63) << 4
                cnt0 = ((tok_base + o * _RG + t_rel) * VOCAB + c0
                        + _K1)
                x1 = lane + cnt0.astype(jnp.uint32)
                buf_v[pl.ds(ci * _NL, _NL)] = _threefry_bits(x1)
            return c2

        jax.lax.fori_loop(0, _GCH // _UNROLL, _chunk, 0)
        pltpu.sync_copy(
            buf_v,
            out_hbm.at[pl.ds((wid * _TPW + o * _RG) * PV, _GELEM)])
        return carry

    jax.lax.fori_loop(0, _NG, _group, 0)


_sc_bits = functools.partial(
    pl.kernel,
    _bits_body,
    out_type=jax.ShapeDtypeStruct((SC_TOK * PV,), jnp.uint32),
    mesh=plsc.VectorSubcoreMesh(core_axis_name="c", subcore_axis_name="s",
                                num_cores=_NC, num_subcores=_NS),
    scratch_types=[
        pltpu.VMEM((_GELEM,), jnp.uint32),
    ],
)()


_BPW = NTOK // _NW
_CH = _BPW // _NL


def _tgt_body(mb_hbm, y_hbm, tabf_hbm, out_hbm, mb_v, y_v, idx_v, val_v,
              acc_v, sem):
    wid = jax.lax.axis_index("s") * _NC + jax.lax.axis_index("c")
    base = wid * _BPW
    pltpu.sync_copy(mb_hbm.at[pl.ds(base, _BPW)], mb_v)
    pltpu.sync_copy(y_hbm.at[pl.ds(base, _BPW)], y_v)

    def _mk_idx(i, carry):
        sl = pl.ds(i * _NL, _NL)
        idx_v[sl] = mb_v[sl] * VOCAB + y_v[sl]
        return carry

    jax.lax.fori_loop(0, _CH, _mk_idx, 0)
    pltpu.async_copy(tabf_hbm.at[idx_v], val_v, sem).wait()

    def _acc(i, acc):
        return acc + val_v[pl.ds(i * _NL, _NL)]

    acc_v[...] = jax.lax.fori_loop(0, _CH, _acc,
                                   jnp.zeros((_NL,), jnp.float32))
    pltpu.sync_copy(acc_v, out_hbm.at[wid])


_tgt_partials = functools.partial(
    pl.kernel,
    _tgt_body,
    out_type=jax.ShapeDtypeStruct((_NW, _NL), jnp.float32),
    mesh=plsc.VectorSubcoreMesh(core_axis_name="c", subcore_axis_name="s",
                                num_cores=_NC, num_subcores=_NS),
    scratch_types=[
        pltpu.VMEM((_BPW,), jnp.int32),
        pltpu.VMEM((_BPW,), jnp.int32),
        pltpu.VMEM((_BPW,), jnp.int32),
        pltpu.VMEM((_BPW,), jnp.float32),
        pltpu.VMEM((_NL,), jnp.float32),
        pltpu.SemaphoreType.DMA,
    ],
)()


@jax.jit
def kernel(mini_batch, y, embed_weight):
    mbf = mini_batch.reshape(NTOK)
    mb_a = mbf[:TOK0].reshape(NSTEPS_A, TB, 1)
    mb_b = mbf[TOK0:].reshape(NSTEPS_B, TB, 1)
    tab = jnp.pad(embed_weight, ((0, PV - VOCAB), (0, 0)))
    tab = jnp.pad(tab, ((0, 0), (0, PV - VOCAB)),
                  constant_values=-1e30)
    tab_hi = tab.astype(jnp.bfloat16)
    tab_lo = (tab - tab_hi.astype(jnp.float32)).astype(jnp.bfloat16)

    cnt_inv = (jnp.arange(TB, dtype=jnp.uint32)[:, None] * jnp.uint32(VOCAB)
               + jnp.arange(PV, dtype=jnp.uint32)[None, :]
               + jnp.uint32(_K1))

    bits_sc = _sc_bits()
    tgt_part = _tgt_partials(mbf, y.reshape(NTOK),
                             embed_weight.reshape(VOCAB * VOCAB))

    yp_a, lse_a = _tc_sample(cnt_inv, mb_a, tab_hi, tab_lo, NSTEPS_A,
                             from_sc_bits=False)
    yp_b, lse_b = _tc_sample(bits_sc.reshape(NSTEPS_B, TB, PV), mb_b,
                             tab_hi, tab_lo, NSTEPS_B, from_sc_bits=True)

    loss = ((lse_a[0, 0] + lse_b[0, 0]) - jnp.sum(tgt_part)) \
        * jnp.float32(1.0 / NTOK)
    yp = jnp.concatenate([yp_a.reshape(TOK0), yp_b.reshape(SC_TOK)])
    return yp, loss
